# baseline (speedup 1.0000x reference)
import jax
import jax.numpy as jnp
from jax.experimental import pallas as pl
from jax.experimental.pallas import tpu as pltpu

B = 4096
D0 = 2048
D1 = 1024
D2 = 512
LATENT = 2048
NUM_MODES = 64
MODE_DIM = 512

TB = 128


def _tables_body(mt_ref, ws_ref, bs_ref, wh_ref, bh_ref, st_ref, ht_ref):
    mt = mt_ref[...]
    dn = (((1,), (1,)), ((), ()))
    st_ref[...] = jax.lax.dot_general(
        mt, ws_ref[...], dn, preferred_element_type=jnp.float32) + bs_ref[...]
    ht_ref[...] = jax.lax.dot_general(
        mt, wh_ref[...], dn, preferred_element_type=jnp.float32) + bh_ref[...]


def _film_tables(mode_table, Ws, bs, Wh, bh):
    return pl.pallas_call(
        _tables_body,
        out_shape=(
            jax.ShapeDtypeStruct((NUM_MODES, LATENT), jnp.float32),
            jax.ShapeDtypeStruct((NUM_MODES, LATENT), jnp.float32),
        ),
    )(mode_table, Ws, bs.reshape(1, LATENT), Wh, bh.reshape(1, LATENT))


def _main_body(x_ref, mod_ref, mode_ref, w0_ref, w1_ref, w2_ref,
               b0_ref, b1_ref, b2_ref, st_ref, ht_ref, out_ref,
               w0b, w1b, w2b, stb, htb):
    i = pl.program_id(0)

    @pl.when(i == 0)
    def _():
        w0b[...] = w0_ref[...].astype(jnp.bfloat16)
        w1b[...] = w1_ref[...].astype(jnp.bfloat16)
        w2b[...] = w2_ref[...].astype(jnp.bfloat16)
        stb[...] = st_ref[...].astype(jnp.bfloat16)
        htb[...] = ht_ref[...].astype(jnp.bfloat16)

    x = x_ref[...].astype(jnp.bfloat16)
    dn = (((1,), (1,)), ((), ()))
    p0 = jax.lax.dot_general(x, w0b[...], dn,
                             preferred_element_type=jnp.float32) + b0_ref[...]
    p1 = jax.lax.dot_general(x[:, :D1], w1b[...], dn,
                             preferred_element_type=jnp.float32) + b1_ref[...]
    p2 = jax.lax.dot_general(x[:, :D2], w2b[...], dn,
                             preferred_element_type=jnp.float32) + b2_ref[...]
    mod = mod_ref[0, 0, :]
    modc = mod[:, None]
    content = jnp.where(modc == 0, p0, jnp.where(modc == 1, p1, p2))

    mode = mode_ref[0, 0, :]
    iota = jax.lax.broadcasted_iota(jnp.int32, (TB, NUM_MODES), 1)
    onehot = (mode[:, None] == iota).astype(jnp.bfloat16)
    dng = (((1,), (0,)), ((), ()))
    s = jax.lax.dot_general(onehot, stb[...], dng,
                            preferred_element_type=jnp.float32)
    h = jax.lax.dot_general(onehot, htb[...], dng,
                            preferred_element_type=jnp.float32)
    out_ref[...] = content * (1.0 + s) + h


@jax.jit
def kernel(features, modality_ids, mode_ids, W0, b0, W1, b1, W2, b2,
           mode_table, Ws, bs, Wh, bh):
    st, ht = _film_tables(mode_table, Ws, bs, Wh, bh)
    mod3 = modality_ids.astype(jnp.int32).reshape(B // TB, 1, TB)
    mode3 = mode_ids.astype(jnp.int32).reshape(B // TB, 1, TB)

    out = pl.pallas_call(
        _main_body,
        grid=(B // TB,),
        in_specs=[
            pl.BlockSpec((TB, D0), lambda i: (i, 0)),
            pl.BlockSpec((1, 1, TB), lambda i: (i, 0, 0)),
            pl.BlockSpec((1, 1, TB), lambda i: (i, 0, 0)),
            pl.BlockSpec((LATENT, D0), lambda i: (0, 0)),
            pl.BlockSpec((LATENT, D1), lambda i: (0, 0)),
            pl.BlockSpec((LATENT, D2), lambda i: (0, 0)),
            pl.BlockSpec((1, LATENT), lambda i: (0, 0)),
            pl.BlockSpec((1, LATENT), lambda i: (0, 0)),
            pl.BlockSpec((1, LATENT), lambda i: (0, 0)),
            pl.BlockSpec((NUM_MODES, LATENT), lambda i: (0, 0)),
            pl.BlockSpec((NUM_MODES, LATENT), lambda i: (0, 0)),
        ],
        out_specs=pl.BlockSpec((TB, LATENT), lambda i: (i, 0)),
        out_shape=jax.ShapeDtypeStruct((B, LATENT), jnp.float32),
        scratch_shapes=[
            pltpu.VMEM((LATENT, D0), jnp.bfloat16),
            pltpu.VMEM((LATENT, D1), jnp.bfloat16),
            pltpu.VMEM((LATENT, D2), jnp.bfloat16),
            pltpu.VMEM((NUM_MODES, LATENT), jnp.bfloat16),
            pltpu.VMEM((NUM_MODES, LATENT), jnp.bfloat16),
        ],
    )(features, mod3, mode3, W0, W1, W2, b0.reshape(1, LATENT),
      b1.reshape(1, LATENT), b2.reshape(1, LATENT), st, ht)
    return out


# speedup vs baseline: 2.0655x; 2.0655x over previous
import jax
import jax.numpy as jnp
from jax.experimental import pallas as pl
from jax.experimental.pallas import tpu as pltpu

B = 4096
D0 = 2048
D1 = 1024
D2 = 512
DZ = D0 + D1 + D2
LATENT = 2048
NUM_MODES = 64
MODE_DIM = 512

TB = 256


def _tables_body(mt_ref, ws_ref, bs_ref, wh_ref, bh_ref, st_ref, ht_ref):
    mt = mt_ref[...]
    dn = (((1,), (1,)), ((), ()))
    st_ref[...] = jax.lax.dot_general(
        mt, ws_ref[...], dn, preferred_element_type=jnp.float32) + bs_ref[...]
    ht_ref[...] = jax.lax.dot_general(
        mt, wh_ref[...], dn, preferred_element_type=jnp.float32) + bh_ref[...]


def _film_tables(mode_table, Ws, bs, Wh, bh):
    return pl.pallas_call(
        _tables_body,
        out_shape=(
            jax.ShapeDtypeStruct((NUM_MODES, LATENT), jnp.float32),
            jax.ShapeDtypeStruct((NUM_MODES, LATENT), jnp.float32),
        ),
    )(mode_table, Ws, bs.reshape(1, LATENT), Wh, bh.reshape(1, LATENT))


def _main_body(x_ref, mod_ref, mode_ref, w0_ref, w1_ref, w2_ref,
               b0_ref, b1_ref, b2_ref, st_ref, ht_ref, out_ref,
               wz, bz, sth, zbuf):
    i = pl.program_id(0)

    @pl.when(i == 0)
    def _():
        wz[:, :D0] = w0_ref[...].astype(jnp.bfloat16)
        wz[:, D0:D0 + D1] = w1_ref[...].astype(jnp.bfloat16)
        wz[:, D0 + D1:] = w2_ref[...].astype(jnp.bfloat16)
        bz[...] = jnp.concatenate([
            b0_ref[...], b1_ref[...], b2_ref[...],
            jnp.zeros((NUM_MODES - 3, LATENT), jnp.float32),
        ], axis=0).astype(jnp.bfloat16)
        sth[:, :LATENT] = st_ref[...].astype(jnp.bfloat16)
        sth[:, LATENT:] = ht_ref[...].astype(jnp.bfloat16)

    xb = x_ref[...].astype(jnp.bfloat16)
    mod = mod_ref[0, 0, :]
    mode = mode_ref[0, 0, :]

    modc = mod[:, None]
    m0 = modc == 0
    m1 = modc == 1
    m2 = modc == 2
    zbuf[:, :D0] = jnp.where(m0, xb, jnp.zeros((), jnp.bfloat16))
    zbuf[:, D0:D0 + D1] = jnp.where(m1, xb[:, :D1], jnp.zeros((), jnp.bfloat16))
    zbuf[:, D0 + D1:] = jnp.where(m2, xb[:, :D2], jnp.zeros((), jnp.bfloat16))
    z = zbuf[...]

    iota64b = jax.lax.broadcasted_iota(jnp.int32, (TB, NUM_MODES), 1)
    oh_mod = (mod[:, None] == iota64b).astype(jnp.bfloat16)
    iota64 = jax.lax.broadcasted_iota(jnp.int32, (TB, NUM_MODES), 1)
    oh_mode = (mode[:, None] == iota64).astype(jnp.bfloat16)

    dnk = (((1,), (1,)), ((), ()))
    dng = (((1,), (0,)), ((), ()))
    content = jax.lax.dot_general(z, wz[...], dnk,
                                  preferred_element_type=jnp.float32)
    content = content + jax.lax.dot_general(oh_mod, bz[...], dng,
                                            preferred_element_type=jnp.float32)
    sh = jax.lax.dot_general(oh_mode, sth[...], dng,
                             preferred_element_type=jnp.float32)
    out_ref[...] = content * (1.0 + sh[:, :LATENT]) + sh[:, LATENT:]


@jax.jit
def kernel(features, modality_ids, mode_ids, W0, b0, W1, b1, W2, b2,
           mode_table, Ws, bs, Wh, bh):
    st, ht = _film_tables(mode_table, Ws, bs, Wh, bh)
    mod3 = modality_ids.astype(jnp.int32).reshape(B // TB, 1, TB)
    mode3 = mode_ids.astype(jnp.int32).reshape(B // TB, 1, TB)

    out = pl.pallas_call(
        _main_body,
        grid=(B // TB,),
        in_specs=[
            pl.BlockSpec((TB, D0), lambda i: (i, 0)),
            pl.BlockSpec((1, 1, TB), lambda i: (i, 0, 0)),
            pl.BlockSpec((1, 1, TB), lambda i: (i, 0, 0)),
            pl.BlockSpec((LATENT, D0), lambda i: (0, 0)),
            pl.BlockSpec((LATENT, D1), lambda i: (0, 0)),
            pl.BlockSpec((LATENT, D2), lambda i: (0, 0)),
            pl.BlockSpec((1, LATENT), lambda i: (0, 0)),
            pl.BlockSpec((1, LATENT), lambda i: (0, 0)),
            pl.BlockSpec((1, LATENT), lambda i: (0, 0)),
            pl.BlockSpec((NUM_MODES, LATENT), lambda i: (0, 0)),
            pl.BlockSpec((NUM_MODES, LATENT), lambda i: (0, 0)),
        ],
        out_specs=pl.BlockSpec((TB, LATENT), lambda i: (i, 0)),
        out_shape=jax.ShapeDtypeStruct((B, LATENT), jnp.float32),
        scratch_shapes=[
            pltpu.VMEM((LATENT, DZ), jnp.bfloat16),
            pltpu.VMEM((NUM_MODES, LATENT), jnp.bfloat16),
            pltpu.VMEM((NUM_MODES, 2 * LATENT), jnp.bfloat16),
            pltpu.VMEM((TB, DZ), jnp.bfloat16),
        ],
    )(features, mod3, mode3, W0, W1, W2, b0.reshape(1, LATENT),
      b1.reshape(1, LATENT), b2.reshape(1, LATENT), st, ht)
    return out
